# Initial kernel scaffold; baseline (speedup 1.0000x reference)
import functools
import math

import jax
import jax.numpy as jnp
from jax import lax
from jax.experimental import pallas as pl
from jax.experimental.pallas import tpu as pltpu
from jax.experimental.pallas import tpu_sc as plsc

VOCAB = 100000
D = 1024
SEQ = 4096
BATCH = 4
B = BATCH * SEQ
SCALE = math.sqrt(D)

NC = 2
NS = 16
NW = NC * NS
ROWS_PER_W = B // NW
CK = 32
NCHUNK = ROWS_PER_W // CK
L = 16
VPR = D // L

_mesh = plsc.VectorSubcoreMesh(
    core_axis_name="c", subcore_axis_name="s", num_cores=NC, num_subcores=NS
)


@functools.partial(
    pl.kernel,
    out_type=jax.ShapeDtypeStruct((B, D), jnp.float32),
    mesh=_mesh,
    scratch_types=[
        pltpu.VMEM((ROWS_PER_W,), jnp.int32),
        pltpu.VMEM((CK, D), jnp.float32),
        pltpu.VMEM((CK, D), jnp.float32),
        pltpu.SemaphoreType.DMA,
    ],
)
def _embed_sc(x_hbm, tok_hbm, pos_hbm, out_hbm, idx_v, rows_v, pos_v, sem):
    wid = lax.axis_index("s") * NC + lax.axis_index("c")
    base = wid * ROWS_PER_W
    pos_base = (wid % (SEQ // ROWS_PER_W)) * ROWS_PER_W

    pltpu.sync_copy(x_hbm.at[pl.ds(base, ROWS_PER_W)], idx_v)

    @pl.loop(0, NCHUNK)
    def _chunk(k):
        off = k * CK
        gather = pltpu.async_copy(
            tok_hbm.at[idx_v.at[pl.ds(off, CK)]], rows_v, sem
        )
        pltpu.sync_copy(pos_hbm.at[pl.ds(pos_base + off, CK)], pos_v)
        gather.wait()

        @pl.loop(0, CK)
        def _row(r):
            for j in range(VPR):
                sl = pl.ds(j * L, L)
                rows_v[r, sl] = rows_v[r, sl] * SCALE + pos_v[r, sl]

        pltpu.sync_copy(rows_v, out_hbm.at[pl.ds(base + off, CK)])


def kernel(x, token_table, pos_table):
    out = _embed_sc(x.reshape(-1), token_table, pos_table)
    return out.reshape(BATCH, SEQ, D)


# speedup vs baseline: 1.1485x; 1.1485x over previous
import functools
import math

import jax
import jax.numpy as jnp
from jax import lax
from jax.experimental import pallas as pl
from jax.experimental.pallas import tpu as pltpu
from jax.experimental.pallas import tpu_sc as plsc

VOCAB = 100000
D = 1024
SEQ = 4096
BATCH = 4
B = BATCH * SEQ
SCALE = math.sqrt(D)

NC = 2
NS = 16
NW = NC * NS
NSEQ_W = SEQ // NW
ROWS_PER_W = B // NW
CK = 32
NCHUNK = ROWS_PER_W // CK
NPAIR = NCHUNK // 2
L = 16
VPR = D // L

_mesh = plsc.VectorSubcoreMesh(
    core_axis_name="c", subcore_axis_name="s", num_cores=NC, num_subcores=NS
)


@functools.partial(
    pl.kernel,
    out_type=jax.ShapeDtypeStruct((B, D), jnp.float32),
    mesh=_mesh,
    scratch_types=[
        pltpu.VMEM((ROWS_PER_W,), jnp.int32),
        pltpu.VMEM((2 * CK, D), jnp.float32),
        pltpu.VMEM((CK, D), jnp.float32),
        pltpu.SemaphoreType.DMA,
        pltpu.SemaphoreType.DMA,
        pltpu.SemaphoreType.DMA,
    ],
)
def _embed_sc(x_hbm, tok_hbm, pos_hbm, out_hbm, idx_v, rows_v, pos_v,
              gsem0, gsem1, ssem):
    wid = lax.axis_index("s") * NC + lax.axis_index("c")
    s0 = wid * NSEQ_W

    for b in range(BATCH):
        pltpu.sync_copy(
            x_hbm.at[pl.ds(b * SEQ + s0, NSEQ_W)],
            idx_v.at[pl.ds(b * NSEQ_W, NSEQ_W)],
        )

    def idx_off(c):
        return (c % BATCH) * NSEQ_W + (c // BATCH) * CK

    def out_off(c):
        return (c % BATCH) * SEQ + s0 + (c // BATCH) * CK

    def issue_gather(c, boff, sem):
        pltpu.async_copy(
            tok_hbm.at[idx_v.at[pl.ds(idx_off(c), CK)]],
            rows_v.at[pl.ds(boff, CK)],
            sem,
        )

    def wait_gather(boff, sem):
        pltpu.make_async_copy(
            tok_hbm.at[pl.ds(0, CK)], rows_v.at[pl.ds(boff, CK)], sem
        ).wait()

    def drain_store():
        pltpu.make_async_copy(
            rows_v.at[pl.ds(0, CK)], out_hbm.at[pl.ds(0, CK)], ssem
        ).wait()

    def compute(boff):
        @pl.loop(0, CK)
        def _row(r):
            for j in range(VPR):
                sl = pl.ds(j * L, L)
                rows_v[boff + r, sl] = rows_v[boff + r, sl] * SCALE + pos_v[r, sl]

    def store(c, boff):
        pltpu.async_copy(
            rows_v.at[pl.ds(boff, CK)], out_hbm.at[pl.ds(out_off(c), CK)], ssem
        )

    issue_gather(0, 0, gsem0)

    @pl.loop(0, NPAIR)
    def _pair(t):
        c0 = 2 * t
        c1 = c0 + 1

        @pl.when(t >= 1)
        def _():
            drain_store()

        issue_gather(c1, CK, gsem1)

        @pl.when(t % 2 == 0)
        def _():
            pltpu.sync_copy(
                pos_hbm.at[pl.ds(s0 + (t // 2) * CK, CK)], pos_v
            )

        wait_gather(0, gsem0)
        compute(0)
        store(c0, 0)

        @pl.when(t < NPAIR - 1)
        def _():
            drain_store()
            issue_gather(c0 + 2, 0, gsem0)

        wait_gather(CK, gsem1)
        compute(CK)
        store(c1, CK)

    drain_store()
    drain_store()


def kernel(x, token_table, pos_table):
    out = _embed_sc(x.reshape(-1), token_table, pos_table)
    return out.reshape(BATCH, SEQ, D)
